# baseline (speedup 1.0000x reference)
import jax
import jax.numpy as jnp
from jax import lax
from jax.experimental import pallas as pl
from jax.experimental.pallas import tpu as pltpu
from jax.experimental.pallas import tpu_sc as plsc

B = 16384
EMBED = 32
N_GENRES = 8
NC = 2
NS = 16
NW = NC * NS
BPW = B // NW
TITLE_ROW = 100001
T_CHUNK = 2048
T_STEPS = B // T_CHUNK


def _body(ttT, genre_tab, mt, mgT, outT,
          trow_v, tidx_v, tout_v, gidx_v, gtab_v, gacc_v, tsem):
    wid = lax.axis_index("s") * NC + lax.axis_index("c")
    base = wid * BPW

    tcp = pltpu.async_copy(ttT.at[pl.ds(wid, 1)], trow_v, tsem)

    pltpu.sync_copy(genre_tab, gtab_v)
    pltpu.sync_copy(mgT.at[:, pl.ds(base, BPW)], gidx_v)

    def genre_body(j, _):
        gvs = [gidx_v[g, pl.ds(j * 16, 16)] for g in range(N_GENRES)]
        for d in range(EMBED):
            dcol = jnp.full((16,), d, dtype=jnp.int32)
            acc = None
            for g in range(N_GENRES):
                val = plsc.load_gather(gtab_v, [gvs[g], dcol])
                acc = val if acc is None else acc + val
            gacc_v[d, pl.ds(j * 16, 16)] = acc * 0.125
        return _
    lax.fori_loop(0, BPW // 16, genre_body, None)

    pltpu.sync_copy(gacc_v, outT.at[pl.ds(EMBED, EMBED), pl.ds(base, BPW)])

    tcp.wait()
    zrow = jnp.zeros((16,), dtype=jnp.int32)
    for q in range(T_STEPS):
        pltpu.sync_copy(mt.at[pl.ds(q * T_CHUNK, T_CHUNK)], tidx_v)

        def title_body(j, _):
            for s in range(8):
                off = j * 128 + s * 16
                iv = tidx_v[pl.ds(off, 16)]
                tout_v[0, pl.ds(off, 16)] = plsc.load_gather(trow_v, [zrow, iv])
            return _
        lax.fori_loop(0, T_CHUNK // 128, title_body, None)
        pltpu.sync_copy(tout_v, outT.at[pl.ds(wid, 1), pl.ds(q * T_CHUNK, T_CHUNK)])


@jax.jit
def _run(ttT, genre_table, mt, mgT):
    mesh = plsc.VectorSubcoreMesh(core_axis_name="c", subcore_axis_name="s",
                                  num_cores=NC, num_subcores=NS)
    return pl.kernel(
        _body,
        out_type=jax.ShapeDtypeStruct((2 * EMBED, B), jnp.float32),
        mesh=mesh,
        scratch_types=[
            pltpu.VMEM((1, TITLE_ROW), jnp.float32),
            pltpu.VMEM((T_CHUNK,), jnp.int32),
            pltpu.VMEM((1, T_CHUNK), jnp.float32),
            pltpu.VMEM((N_GENRES, BPW), jnp.int32),
            pltpu.VMEM((21, EMBED), jnp.float32),
            pltpu.VMEM((EMBED, BPW), jnp.float32),
            pltpu.SemaphoreType.DMA,
        ],
        compiler_params=pltpu.CompilerParams(use_tc_tiling_on_sc=False,
                                             needs_layout_passes=False),
    )(ttT, genre_table, mt, mgT)


def kernel(title_table, genre_table, movie_title, movie_genres):
    outT = _run(title_table.T, genre_table,
                movie_title.astype(jnp.int32),
                movie_genres.astype(jnp.int32).T)
    return outT.T


# speedup vs baseline: 1.0541x; 1.0541x over previous
import jax
import jax.numpy as jnp
from jax import lax
from jax.experimental import pallas as pl
from jax.experimental.pallas import tpu as pltpu
from jax.experimental.pallas import tpu_sc as plsc

B = 16384
EMBED = 32
N_GENRES = 8
NC = 2
NS = 16
NW = NC * NS
BPW = B // NW
TITLE_ROW = 100001
T_HALF = B // 2
T_CHUNK = 2048


def _body(ttT, gtab_flat, mt, mgT, outT,
          trow_v, tbuf_v, gidx_v, gtab_v, gacc_v,
          trow_sem, ti_sem, to_sem, gi_sem, go_sem):
    wid = lax.axis_index("s") * NC + lax.axis_index("c")
    base = wid * BPW

    trow_cp = pltpu.async_copy(ttT.at[pl.ds(wid, 1)], trow_v, trow_sem)
    ti_cp = pltpu.async_copy(mt.at[pl.ds(0, T_HALF)], tbuf_v.at[0], ti_sem)
    gidx_cp = pltpu.async_copy(mgT.at[:, pl.ds(base, BPW)], gidx_v, gi_sem)
    pltpu.sync_copy(gtab_flat, gtab_v)

    gidx_cp.wait()

    def genre_body(j, _):
        gsh = [gidx_v[g, pl.ds(j * 16, 16)] * jnp.int32(EMBED)
               for g in range(N_GENRES)]
        for d in range(EMBED):
            acc = None
            for g in range(N_GENRES):
                val = plsc.load_gather(gtab_v, [gsh[g] + jnp.int32(d)])
                acc = val if acc is None else acc + val
            gacc_v[d, pl.ds(j * 16, 16)] = acc * 0.125
        return _
    lax.fori_loop(0, BPW // 16, genre_body, None)

    pltpu.async_copy(gacc_v, outT.at[pl.ds(EMBED, EMBED), pl.ds(base, BPW)],
                     go_sem)

    trow_cp.wait()
    zrow = jnp.zeros((16,), dtype=jnp.int32)
    for h in range(2):
        hbase = h * T_HALF
        ti_cp.wait()
        for q in range(T_HALF // T_CHUNK):

            def title_body(j, _, q=q):
                for s in range(8):
                    off = q * T_CHUNK + j * 128 + s * 16
                    iv = plsc.bitcast(tbuf_v[0, pl.ds(off, 16)], jnp.int32)
                    tbuf_v[0, pl.ds(off, 16)] = plsc.load_gather(
                        trow_v, [zrow, iv])
                return _
            lax.fori_loop(0, T_CHUNK // 128, title_body, None)
            pltpu.async_copy(
                tbuf_v.at[:, pl.ds(q * T_CHUNK, T_CHUNK)],
                outT.at[pl.ds(wid, 1), pl.ds(hbase + q * T_CHUNK, T_CHUNK)],
                to_sem)

        for q in range(T_HALF // T_CHUNK):
            pltpu.make_async_copy(
                tbuf_v.at[:, pl.ds(q * T_CHUNK, T_CHUNK)],
                outT.at[pl.ds(wid, 1), pl.ds(hbase + q * T_CHUNK, T_CHUNK)],
                to_sem).wait()
        if h == 0:
            ti_cp = pltpu.async_copy(mt.at[pl.ds(T_HALF, T_HALF)],
                                     tbuf_v.at[0], ti_sem)

    pltpu.make_async_copy(
        gacc_v, outT.at[pl.ds(EMBED, EMBED), pl.ds(base, BPW)], go_sem).wait()


@jax.jit
def _run(ttT, gtab_flat, mt_f32, mgT):
    mesh = plsc.VectorSubcoreMesh(core_axis_name="c", subcore_axis_name="s",
                                  num_cores=NC, num_subcores=NS)
    return pl.kernel(
        _body,
        out_type=jax.ShapeDtypeStruct((2 * EMBED, B), jnp.float32),
        mesh=mesh,
        scratch_types=[
            pltpu.VMEM((1, TITLE_ROW), jnp.float32),
            pltpu.VMEM((1, T_HALF), jnp.float32),
            pltpu.VMEM((N_GENRES, BPW), jnp.int32),
            pltpu.VMEM((21 * EMBED,), jnp.float32),
            pltpu.VMEM((EMBED, BPW), jnp.float32),
            pltpu.SemaphoreType.DMA,
            pltpu.SemaphoreType.DMA,
            pltpu.SemaphoreType.DMA,
            pltpu.SemaphoreType.DMA,
            pltpu.SemaphoreType.DMA,
        ],
        compiler_params=pltpu.CompilerParams(use_tc_tiling_on_sc=False,
                                             needs_layout_passes=False),
    )(ttT, gtab_flat, mt_f32, mgT)


def kernel(title_table, genre_table, movie_title, movie_genres):
    mt_f32 = jax.lax.bitcast_convert_type(movie_title.astype(jnp.int32),
                                          jnp.float32)
    outT = _run(title_table.T, genre_table.reshape(-1), mt_f32,
                movie_genres.astype(jnp.int32).T)
    return outT.T


# speedup vs baseline: 1.9578x; 1.8574x over previous
import jax
import jax.numpy as jnp
from jax import lax
from jax.experimental import pallas as pl
from jax.experimental.pallas import tpu as pltpu
from jax.experimental.pallas import tpu_sc as plsc

B = 16384
EMBED = 32
N_GENRES = 8
NC = 2
NS = 16
NW = NC * NS
BPW = B // NW
TITLE_ROW = 100001
T_HALF = B // 2
T_CHUNK = 2048


def _body(ttT, gtab_flat, mt, mgT, outT,
          trow_v, tbuf_v, gidx_v, gtab_v, gacc_v,
          trow_sem, ti_sem, to_sem, gi_sem, go_sem):
    wid = lax.axis_index("s") * NC + lax.axis_index("c")
    base = wid * BPW

    trow_cp = pltpu.async_copy(ttT.at[pl.ds(wid, 1)], trow_v, trow_sem)
    ti_cp = pltpu.async_copy(mt.at[pl.ds(0, T_HALF)], tbuf_v.at[0], ti_sem)
    gidx_cp = pltpu.async_copy(mgT.at[:, pl.ds(base, BPW)], gidx_v, gi_sem)
    pltpu.sync_copy(gtab_flat, gtab_v)

    gidx_cp.wait()

    def genre_body(j, _):
        gvs = [gidx_v[g, pl.ds(j * 16, 16)] for g in range(N_GENRES)]
        for d in range(EMBED):
            drow = jnp.full((16,), d, dtype=jnp.int32)
            acc = None
            for g in range(N_GENRES):
                val = plsc.load_gather(gtab_v, [drow, gvs[g]])
                acc = val if acc is None else acc + val
            gacc_v[d, pl.ds(j * 16, 16)] = acc * 0.125
        return _
    lax.fori_loop(0, BPW // 16, genre_body, None)

    pltpu.async_copy(gacc_v, outT.at[pl.ds(EMBED, EMBED), pl.ds(base, BPW)],
                     go_sem)

    trow_cp.wait()
    zrow = jnp.zeros((16,), dtype=jnp.int32)
    for h in range(2):
        hbase = h * T_HALF
        ti_cp.wait()
        for q in range(T_HALF // T_CHUNK):

            def title_body(j, _, q=q):
                for s in range(8):
                    off = q * T_CHUNK + j * 128 + s * 16
                    iv = plsc.bitcast(tbuf_v[0, pl.ds(off, 16)], jnp.int32)
                    tbuf_v[0, pl.ds(off, 16)] = plsc.load_gather(
                        trow_v, [zrow, iv])
                return _
            lax.fori_loop(0, T_CHUNK // 128, title_body, None)
            pltpu.async_copy(
                tbuf_v.at[:, pl.ds(q * T_CHUNK, T_CHUNK)],
                outT.at[pl.ds(wid, 1), pl.ds(hbase + q * T_CHUNK, T_CHUNK)],
                to_sem)

        for q in range(T_HALF // T_CHUNK):
            pltpu.make_async_copy(
                tbuf_v.at[:, pl.ds(q * T_CHUNK, T_CHUNK)],
                outT.at[pl.ds(wid, 1), pl.ds(hbase + q * T_CHUNK, T_CHUNK)],
                to_sem).wait()
        if h == 0:
            ti_cp = pltpu.async_copy(mt.at[pl.ds(T_HALF, T_HALF)],
                                     tbuf_v.at[0], ti_sem)

    pltpu.make_async_copy(
        gacc_v, outT.at[pl.ds(EMBED, EMBED), pl.ds(base, BPW)], go_sem).wait()


@jax.jit
def _run(ttT, gtab_flat, mt_f32, mgT):
    mesh = plsc.VectorSubcoreMesh(core_axis_name="c", subcore_axis_name="s",
                                  num_cores=NC, num_subcores=NS)
    return pl.kernel(
        _body,
        out_type=jax.ShapeDtypeStruct((2 * EMBED, B), jnp.float32),
        mesh=mesh,
        scratch_types=[
            pltpu.VMEM((1, TITLE_ROW), jnp.float32),
            pltpu.VMEM((1, T_HALF), jnp.float32),
            pltpu.VMEM((N_GENRES, BPW), jnp.int32),
            pltpu.VMEM((EMBED, 21), jnp.float32),
            pltpu.VMEM((EMBED, BPW), jnp.float32),
            pltpu.SemaphoreType.DMA,
            pltpu.SemaphoreType.DMA,
            pltpu.SemaphoreType.DMA,
            pltpu.SemaphoreType.DMA,
            pltpu.SemaphoreType.DMA,
        ],
        compiler_params=pltpu.CompilerParams(use_tc_tiling_on_sc=False,
                                             needs_layout_passes=False),
    )(ttT, gtab_flat, mt_f32, mgT)


def kernel(title_table, genre_table, movie_title, movie_genres):
    mt_f32 = jax.lax.bitcast_convert_type(movie_title.astype(jnp.int32),
                                          jnp.float32)
    outT = _run(title_table.T, genre_table.T, mt_f32,
                movie_genres.astype(jnp.int32).T)
    return outT.T
